# baseline (speedup 1.0000x reference)
import functools
import math

import jax
import jax.numpy as jnp
from jax import lax
from jax.experimental import pallas as pl
from jax.experimental.pallas import tpu as pltpu
from jax.experimental.pallas import tpu_sc as plsc

B = 2
P = 512
T = B * P
D_MODEL = 768
N_HEADS = 12
D_HEAD = D_MODEL // N_HEADS
D_FF = 3072
N_EXPERTS = 4
M_FEATS = int(D_HEAD * math.log(D_HEAD))
M_PAD = 384
HEADS_PER_BLK = 2
CIN_FLAT = 128
F_BLK = 768
N_FBLK = D_FF // F_BLK
BT = 256
NBLK = 12
SROWS = NBLK * BT
NC, NS = 2, 16
NW = NC * NS


def _embed_qkv_body(xf_ref, wemb_ref, wq_ref, bq_ref, wk_ref, bk_ref,
                    wv_ref, bv_ref, h_ref, qn_ref, kn_ref, vn_ref):
    h = jnp.dot(xf_ref[...], wemb_ref[...], preferred_element_type=jnp.float32)
    h_ref[...] = h
    dn = 1.0 / (float(D_HEAD) ** 0.25)
    qn_ref[...] = (jnp.dot(h, wq_ref[...], preferred_element_type=jnp.float32)
                   + bq_ref[...]) * dn
    kn_ref[...] = (jnp.dot(h, wk_ref[...], preferred_element_type=jnp.float32)
                   + bk_ref[...]) * dn
    vn_ref[...] = jnp.dot(h, wv_ref[...], preferred_element_type=jnp.float32) + bv_ref[...]


def _performer_one_head(qb, kb, vb, om):
    dims = (((1,), (1,)), ((), ()))
    pq = jax.lax.dot_general(qb, om, dims, preferred_element_type=jnp.float32)
    pk = jax.lax.dot_general(kb, om, dims, preferred_element_type=jnp.float32)
    lane = jax.lax.broadcasted_iota(jnp.int32, (P, M_PAD), 1)
    valid = lane < M_FEATS
    dq = 0.5 * jnp.sum(qb * qb, axis=-1, keepdims=True)
    dk = 0.5 * jnp.sum(kb * kb, axis=-1, keepdims=True)
    mq = jnp.max(pq, axis=-1, keepdims=True)
    mk = jnp.max(pk)
    inv_sqrt_m = jnp.float32(1.0 / math.sqrt(M_FEATS))
    eps = jnp.float32(1e-6)
    qp = jnp.exp(pq - (dq + mq)) * inv_sqrt_m + eps
    kp = jnp.where(valid, jnp.exp(pk - (dk + mk)) * inv_sqrt_m + eps, 0.0)
    kv = jax.lax.dot_general(kp, vb, (((0,), (0,)), ((), ())),
                             preferred_element_type=jnp.float32)
    ksum = jnp.sum(kp, axis=0, keepdims=True)
    z = jax.lax.dot_general(qp, ksum, (((1,), (1,)), ((), ())),
                            preferred_element_type=jnp.float32)
    return jnp.dot(qp, kv, preferred_element_type=jnp.float32) / z


def _performer_body(qn_ref, kn_ref, vn_ref, om_ref, o_ref):
    om = om_ref[...]
    outs = []
    for hh in range(HEADS_PER_BLK):
        sl = (slice(None), slice(hh * D_HEAD, (hh + 1) * D_HEAD))
        outs.append(_performer_one_head(
            qn_ref[0][sl], kn_ref[0][sl], vn_ref[0][sl], om))
    o_ref[0] = jnp.concatenate(outs, axis=1)


def _layernorm(s, g, b):
    mu = jnp.mean(s, axis=-1, keepdims=True)
    var = jnp.mean((s - mu) ** 2, axis=-1, keepdims=True)
    return (s - mu) * jax.lax.rsqrt(var + 1e-5) * g + b


def _proj_router_body(attn_ref, wo_ref, bo_ref, h_ref, g2_ref, be2_ref,
                      wg_ref, bg_ref, h2_ref, gates_ref):
    a = jnp.dot(attn_ref[...], wo_ref[...], preferred_element_type=jnp.float32) + bo_ref[...]
    h2 = _layernorm(h_ref[...] + a, g2_ref[...], be2_ref[...])
    h2_ref[...] = h2

    logits = jnp.dot(h2, wg_ref[...], preferred_element_type=jnp.float32) + bg_ref[...]
    lane = jax.lax.broadcasted_iota(jnp.int32, (T, 128), 1)
    logits = jnp.where(lane < N_EXPERTS, logits, jnp.float32(-1e30))
    big = jnp.int32(10**6)
    m1 = jnp.max(logits, axis=-1, keepdims=True)
    i1 = jnp.min(jnp.where(logits == m1, lane, big), axis=-1, keepdims=True)
    first1 = lane == i1
    l2 = jnp.where(first1, jnp.float32(-1e30), logits)
    m2 = jnp.max(l2, axis=-1, keepdims=True)
    i2 = jnp.min(jnp.where(l2 == m2, lane, big), axis=-1, keepdims=True)
    first2 = lane == i2
    ex = jnp.exp(m2 - m1)
    g1 = 1.0 / (1.0 + ex)
    g2v = ex / (1.0 + ex)
    gates_ref[...] = jnp.where(first1, g1, 0.0) + jnp.where(first2, g2v, 0.0)


def _moe_dense_body(h2_ref, gates_ref, w1_ref, b1_ref, w2_ref, b2_ref,
                    g3_ref, be3_ref, out_ref, acc_ref, eacc_ref):
    e = pl.program_id(0)
    f = pl.program_id(1)

    @pl.when((e == 0) & (f == 0))
    def _init():
        acc_ref[...] = h2_ref[...]

    hb = jnp.maximum(
        jnp.dot(h2_ref[...], w1_ref[0], preferred_element_type=jnp.float32)
        + b1_ref[0], 0.0)
    part = jnp.dot(hb, w2_ref[0], preferred_element_type=jnp.float32)

    @pl.when(f == 0)
    def _estart():
        eacc_ref[...] = part

    @pl.when((f > 0) & (f < N_FBLK - 1))
    def _emid():
        eacc_ref[...] += part

    @pl.when(f == N_FBLK - 1)
    def _efin():
        lane = jax.lax.broadcasted_iota(jnp.int32, (T, 128), 1)
        ge = jnp.sum(jnp.where(lane == e, gates_ref[...], 0.0),
                     axis=-1, keepdims=True)
        acc_ref[...] += ge * (eacc_ref[...] + part + b2_ref[0])

    @pl.when((e == N_EXPERTS - 1) & (f == N_FBLK - 1))
    def _final():
        out_ref[...] = _layernorm(acc_ref[...], g3_ref[...], be3_ref[...])


def kernel(x, W_emb, Wq, bq, Wk, bk, Wv, bv, Wo, bo, omega, Wg, bg,
           W1, b1, W2, b2, g2, be2, g3, be3):
    xf = x.reshape(T, -1)
    om_pad = jnp.zeros((M_PAD, D_HEAD), jnp.float32).at[:M_FEATS].set(omega)
    wg_pad = jnp.zeros((D_MODEL, 128), jnp.float32).at[:, :N_EXPERTS].set(Wg)
    bg_pad = jnp.zeros((1, 128), jnp.float32).at[0, :N_EXPERTS].set(bg)
    r2 = lambda a: a.reshape(1, -1)

    h, qn, kn, vn = pl.pallas_call(
        _embed_qkv_body,
        out_shape=[jax.ShapeDtypeStruct((T, D_MODEL), jnp.float32)] * 4,
    )(xf, W_emb, Wq, r2(bq), Wk, r2(bk), Wv, r2(bv))

    head_spec = pl.BlockSpec((1, P, HEADS_PER_BLK * D_HEAD),
                             lambda b, hh: (b, 0, hh))
    attn = pl.pallas_call(
        _performer_body,
        grid=(B, N_HEADS // HEADS_PER_BLK),
        in_specs=[head_spec, head_spec, head_spec,
                  pl.BlockSpec((M_PAD, D_HEAD), lambda b, hh: (0, 0))],
        out_specs=head_spec,
        out_shape=jax.ShapeDtypeStruct((B, P, D_MODEL), jnp.float32),
    )(qn.reshape(B, P, D_MODEL), kn.reshape(B, P, D_MODEL),
      vn.reshape(B, P, D_MODEL), om_pad)

    h2, gates = pl.pallas_call(
        _proj_router_body,
        out_shape=[jax.ShapeDtypeStruct((T, D_MODEL), jnp.float32),
                   jax.ShapeDtypeStruct((T, 128), jnp.float32)],
    )(attn.reshape(T, D_MODEL), Wo, r2(bo), h, r2(g2), r2(be2), wg_pad, bg_pad)

    full = lambda shape: pl.BlockSpec(shape, lambda e, f: (0,) * len(shape))
    out = pl.pallas_call(
        _moe_dense_body,
        grid=(N_EXPERTS, N_FBLK),
        in_specs=[
            full((T, D_MODEL)),
            full((T, 128)),
            pl.BlockSpec((1, D_MODEL, F_BLK), lambda e, f: (e, 0, f)),
            pl.BlockSpec((1, 1, F_BLK), lambda e, f: (e, 0, f)),
            pl.BlockSpec((1, F_BLK, D_MODEL), lambda e, f: (e, f, 0)),
            pl.BlockSpec((1, 1, D_MODEL), lambda e, f: (e, 0, 0)),
            full((1, D_MODEL)),
            full((1, D_MODEL)),
        ],
        out_specs=full((T, D_MODEL)),
        out_shape=jax.ShapeDtypeStruct((T, D_MODEL), jnp.float32),
        scratch_shapes=[pltpu.VMEM((T, D_MODEL), jnp.float32),
                        pltpu.VMEM((T, D_MODEL), jnp.float32)],
    )(h2, gates, W1, b1.reshape(N_EXPERTS, 1, D_FF), W2,
      b2.reshape(N_EXPERTS, 1, D_MODEL), r2(g3), r2(be3))

    return out.reshape(B, 1, P, D_MODEL)


# speedup vs baseline: 1.0337x; 1.0337x over previous
import functools
import math

import jax
import jax.numpy as jnp
from jax import lax
from jax.experimental import pallas as pl
from jax.experimental.pallas import tpu as pltpu
from jax.experimental.pallas import tpu_sc as plsc

B = 2
P = 512
T = B * P
D_MODEL = 768
N_HEADS = 12
D_HEAD = D_MODEL // N_HEADS
D_FF = 3072
N_EXPERTS = 4
M_FEATS = int(D_HEAD * math.log(D_HEAD))
M_PAD = 384
HEADS_PER_BLK = 2
CIN_FLAT = 128
F_BLK = 1536
N_FBLK = D_FF // F_BLK
BT = 256
NBLK = 12
SROWS = NBLK * BT
NC, NS = 2, 16
NW = NC * NS


def _embed_qkv_body(xf_ref, wemb_ref, wq_ref, bq_ref, wk_ref, bk_ref,
                    wv_ref, bv_ref, h_ref, qn_ref, kn_ref, vn_ref):
    h = jnp.dot(xf_ref[...], wemb_ref[...], preferred_element_type=jnp.float32)
    h_ref[...] = h
    dn = 1.0 / (float(D_HEAD) ** 0.25)
    qn_ref[...] = (jnp.dot(h, wq_ref[...], preferred_element_type=jnp.float32)
                   + bq_ref[...]) * dn
    kn_ref[...] = (jnp.dot(h, wk_ref[...], preferred_element_type=jnp.float32)
                   + bk_ref[...]) * dn
    vn_ref[...] = jnp.dot(h, wv_ref[...], preferred_element_type=jnp.float32) + bv_ref[...]


def _performer_one_head(qb, kb, vb, om):
    dims = (((1,), (1,)), ((), ()))
    pq = jax.lax.dot_general(qb, om, dims, preferred_element_type=jnp.float32)
    pk = jax.lax.dot_general(kb, om, dims, preferred_element_type=jnp.float32)
    lane = jax.lax.broadcasted_iota(jnp.int32, (P, M_PAD), 1)
    valid = lane < M_FEATS
    dq = 0.5 * jnp.sum(qb * qb, axis=-1, keepdims=True)
    dk = 0.5 * jnp.sum(kb * kb, axis=-1, keepdims=True)
    mq = jnp.max(pq, axis=-1, keepdims=True)
    mk = jnp.max(pk)
    inv_sqrt_m = jnp.float32(1.0 / math.sqrt(M_FEATS))
    eps = jnp.float32(1e-6)
    qp = jnp.exp(pq - (dq + mq)) * inv_sqrt_m + eps
    kp = jnp.where(valid, jnp.exp(pk - (dk + mk)) * inv_sqrt_m + eps, 0.0)
    kv = jax.lax.dot_general(kp, vb, (((0,), (0,)), ((), ())),
                             preferred_element_type=jnp.float32)
    ksum = jnp.sum(kp, axis=0, keepdims=True)
    z = jax.lax.dot_general(qp, ksum, (((1,), (1,)), ((), ())),
                            preferred_element_type=jnp.float32)
    return jnp.dot(qp, kv, preferred_element_type=jnp.float32) / z


def _performer_body(qn_ref, kn_ref, vn_ref, om_ref, o_ref):
    om = om_ref[...]
    outs = []
    for hh in range(HEADS_PER_BLK):
        sl = (slice(None), slice(hh * D_HEAD, (hh + 1) * D_HEAD))
        outs.append(_performer_one_head(
            qn_ref[0][sl], kn_ref[0][sl], vn_ref[0][sl], om))
    o_ref[0] = jnp.concatenate(outs, axis=1)


def _layernorm(s, g, b):
    mu = jnp.mean(s, axis=-1, keepdims=True)
    var = jnp.mean((s - mu) ** 2, axis=-1, keepdims=True)
    return (s - mu) * jax.lax.rsqrt(var + 1e-5) * g + b


def _proj_router_body(attn_ref, wo_ref, bo_ref, h_ref, g2_ref, be2_ref,
                      wg_ref, bg_ref, h2_ref, gates_ref):
    a = jnp.dot(attn_ref[...], wo_ref[...], preferred_element_type=jnp.float32) + bo_ref[...]
    h2 = _layernorm(h_ref[...] + a, g2_ref[...], be2_ref[...])
    h2_ref[...] = h2

    logits = jnp.dot(h2, wg_ref[...], preferred_element_type=jnp.float32) + bg_ref[...]
    lane = jax.lax.broadcasted_iota(jnp.int32, (T, 128), 1)
    logits = jnp.where(lane < N_EXPERTS, logits, jnp.float32(-1e30))
    big = jnp.int32(10**6)
    m1 = jnp.max(logits, axis=-1, keepdims=True)
    i1 = jnp.min(jnp.where(logits == m1, lane, big), axis=-1, keepdims=True)
    first1 = lane == i1
    l2 = jnp.where(first1, jnp.float32(-1e30), logits)
    m2 = jnp.max(l2, axis=-1, keepdims=True)
    i2 = jnp.min(jnp.where(l2 == m2, lane, big), axis=-1, keepdims=True)
    first2 = lane == i2
    ex = jnp.exp(m2 - m1)
    g1 = 1.0 / (1.0 + ex)
    g2v = ex / (1.0 + ex)
    gates_ref[...] = jnp.where(first1, g1, 0.0) + jnp.where(first2, g2v, 0.0)


def _moe_dense_body(h2_ref, gates_ref, w1_ref, b1_ref, w2_ref, b2_ref,
                    g3_ref, be3_ref, out_ref, acc_ref, eacc_ref):
    e = pl.program_id(0)
    f = pl.program_id(1)

    @pl.when((e == 0) & (f == 0))
    def _init():
        acc_ref[...] = h2_ref[...]

    hb = jnp.maximum(
        jnp.dot(h2_ref[...], w1_ref[0], preferred_element_type=jnp.float32)
        + b1_ref[0], 0.0)
    part = jnp.dot(hb, w2_ref[0], preferred_element_type=jnp.float32)

    @pl.when(f == 0)
    def _estart():
        eacc_ref[...] = part

    @pl.when((f > 0) & (f < N_FBLK - 1))
    def _emid():
        eacc_ref[...] += part

    @pl.when(f == N_FBLK - 1)
    def _efin():
        lane = jax.lax.broadcasted_iota(jnp.int32, (T, 128), 1)
        ge = jnp.sum(jnp.where(lane == e, gates_ref[...], 0.0),
                     axis=-1, keepdims=True)
        acc_ref[...] += ge * (eacc_ref[...] + part + b2_ref[0])

    @pl.when((e == N_EXPERTS - 1) & (f == N_FBLK - 1))
    def _final():
        out_ref[...] = _layernorm(acc_ref[...], g3_ref[...], be3_ref[...])


def kernel(x, W_emb, Wq, bq, Wk, bk, Wv, bv, Wo, bo, omega, Wg, bg,
           W1, b1, W2, b2, g2, be2, g3, be3):
    xf = x.reshape(T, -1)
    om_pad = jnp.zeros((M_PAD, D_HEAD), jnp.float32).at[:M_FEATS].set(omega)
    wg_pad = jnp.zeros((D_MODEL, 128), jnp.float32).at[:, :N_EXPERTS].set(Wg)
    bg_pad = jnp.zeros((1, 128), jnp.float32).at[0, :N_EXPERTS].set(bg)
    r2 = lambda a: a.reshape(1, -1)

    h, qn, kn, vn = pl.pallas_call(
        _embed_qkv_body,
        out_shape=[jax.ShapeDtypeStruct((T, D_MODEL), jnp.float32)] * 4,
    )(xf, W_emb, Wq, r2(bq), Wk, r2(bk), Wv, r2(bv))

    head_spec = pl.BlockSpec((1, P, HEADS_PER_BLK * D_HEAD),
                             lambda b, hh: (b, 0, hh))
    attn = pl.pallas_call(
        _performer_body,
        grid=(B, N_HEADS // HEADS_PER_BLK),
        in_specs=[head_spec, head_spec, head_spec,
                  pl.BlockSpec((M_PAD, D_HEAD), lambda b, hh: (0, 0))],
        out_specs=head_spec,
        out_shape=jax.ShapeDtypeStruct((B, P, D_MODEL), jnp.float32),
    )(qn.reshape(B, P, D_MODEL), kn.reshape(B, P, D_MODEL),
      vn.reshape(B, P, D_MODEL), om_pad)

    h2, gates = pl.pallas_call(
        _proj_router_body,
        out_shape=[jax.ShapeDtypeStruct((T, D_MODEL), jnp.float32),
                   jax.ShapeDtypeStruct((T, 128), jnp.float32)],
    )(attn.reshape(T, D_MODEL), Wo, r2(bo), h, r2(g2), r2(be2), wg_pad, bg_pad)

    full = lambda shape: pl.BlockSpec(shape, lambda e, f: (0,) * len(shape))
    out = pl.pallas_call(
        _moe_dense_body,
        grid=(N_EXPERTS, N_FBLK),
        in_specs=[
            full((T, D_MODEL)),
            full((T, 128)),
            pl.BlockSpec((1, D_MODEL, F_BLK), lambda e, f: (e, 0, f)),
            pl.BlockSpec((1, 1, F_BLK), lambda e, f: (e, 0, f)),
            pl.BlockSpec((1, F_BLK, D_MODEL), lambda e, f: (e, f, 0)),
            pl.BlockSpec((1, 1, D_MODEL), lambda e, f: (e, 0, 0)),
            full((1, D_MODEL)),
            full((1, D_MODEL)),
        ],
        out_specs=full((T, D_MODEL)),
        out_shape=jax.ShapeDtypeStruct((T, D_MODEL), jnp.float32),
        scratch_shapes=[pltpu.VMEM((T, D_MODEL), jnp.float32),
                        pltpu.VMEM((T, D_MODEL), jnp.float32)],
    )(h2, gates, W1, b1.reshape(N_EXPERTS, 1, D_FF), W2,
      b2.reshape(N_EXPERTS, 1, D_MODEL), r2(g3), r2(be3))

    return out.reshape(B, 1, P, D_MODEL)
